# baseline (speedup 1.0000x reference)
import functools

import jax
import jax.numpy as jnp
from jax import lax
from jax.experimental import pallas as pl
from jax.experimental.pallas import tpu as pltpu
from jax.experimental.pallas import tpu_sc as plsc

VOCAB = 168960
HIDDEN = 4096
N_TOK = 4 * 8192

NC = 2
NS = 16
NW = NC * NS
B_PER_W = N_TOK // NW
CHUNK = 8
N_CHUNKS = B_PER_W // CHUNK
NBUF = 3
MAIN = (N_CHUNKS // NBUF) * NBUF
TAIL = N_CHUNKS - MAIN

_mesh = plsc.VectorSubcoreMesh(core_axis_name="c", subcore_axis_name="s")


@functools.partial(
    pl.kernel,
    mesh=_mesh,
    out_type=jax.ShapeDtypeStruct((N_TOK, HIDDEN), jnp.float32),
    scratch_types=[
        pltpu.VMEM((B_PER_W,), jnp.int32),
        pltpu.VMEM((NBUF, CHUNK, HIDDEN), jnp.float32),
        pltpu.SemaphoreType.DMA((NBUF,)),
        pltpu.SemaphoreType.DMA((NBUF,)),
    ],
)
def _embed_sc(ids_hbm, tab_hbm, out_hbm, idx_v, rows_v, sem_g, sem_s):
    wid = lax.axis_index("s") * NC + lax.axis_index("c")
    base = wid * B_PER_W
    pltpu.sync_copy(ids_hbm.at[pl.ds(base, B_PER_W)], idx_v)

    def gather(g, b):
        return pltpu.make_async_copy(
            tab_hbm.at[idx_v.at[pl.ds(g * CHUNK, CHUNK)]],
            rows_v.at[b],
            sem_g.at[b],
        )

    def scatter(g, b):
        return pltpu.make_async_copy(
            rows_v.at[b],
            out_hbm.at[pl.ds(base + g * CHUNK, CHUNK)],
            sem_s.at[b],
        )

    gather(0, 0).start()
    gather(1, 1).start()

    def step(g0, carry):
        for b in range(NBUF):
            g = g0 * NBUF + b
            gather(g, b).wait()
            scatter(g, b).start()

            bp = (b - 1) % NBUF

            @pl.when(g >= 1)
            def _():
                scatter(g - 1, bp).wait()

            @pl.when(g + 2 < N_CHUNKS)
            def _():
                gather(g + 2, bp).start()

        return carry

    lax.fori_loop(0, MAIN // NBUF, step, 0)

    for b in range(TAIL):
        g = MAIN + b
        gather(g, b).wait()
        scatter(g, b).start()
        scatter(g - 1, (b - 1) % NBUF).wait()

    scatter(N_CHUNKS - 1, (N_CHUNKS - 1) % NBUF).wait()


def kernel(input_ids, word_embeddings):
    ids = input_ids.reshape(-1).astype(jnp.int32)
    out = _embed_sc(ids, word_embeddings)
    return out.reshape(input_ids.shape + (word_embeddings.shape[1],))
